# baseline (speedup 1.0000x reference)
import functools

import jax
import jax.numpy as jnp
from jax import lax
from jax.experimental import pallas as pl
from jax.experimental.pallas import tpu as pltpu
from jax.experimental.pallas import tpu_sc as plsc

VOCAB = 100000
EMB_DIM = 64
BATCH = 4096
LANGNUM = 2

_info = plsc.get_sparse_core_info()
_NC, _NS = _info.num_cores, _info.num_subcores
_NW = _NC * _NS
_BPW = BATCH // _NW

_mesh = plsc.VectorSubcoreMesh(core_axis_name="c", subcore_axis_name="s")


@functools.partial(
    pl.kernel,
    mesh=_mesh,
    out_type=jax.ShapeDtypeStruct((LANGNUM, BATCH, EMB_DIM), jnp.float32),
    scratch_types=[
        pltpu.VMEM((_BPW,), jnp.int32),
        pltpu.VMEM((_BPW,), jnp.int32),
        pltpu.VMEM((_BPW, EMB_DIM), jnp.float32),
        pltpu.VMEM((_BPW, EMB_DIM), jnp.float32),
        pltpu.SemaphoreType.DMA,
        pltpu.SemaphoreType.DMA,
    ],
    compiler_params=pltpu.CompilerParams(use_tc_tiling_on_sc=False),
)
def _embed_sc(idx0_hbm, idx1_hbm, emb0_hbm, emb1_hbm, out_hbm,
              idx0_v, idx1_v, rows0_v, rows1_v, sem0, sem1):
    wid = lax.axis_index("s") * _NC + lax.axis_index("c")
    base = wid * _BPW
    pltpu.sync_copy(idx0_hbm.at[pl.ds(base, _BPW)], idx0_v)
    pltpu.sync_copy(idx1_hbm.at[pl.ds(base, _BPW)], idx1_v)
    cp0 = pltpu.async_copy(emb0_hbm.at[idx0_v], rows0_v, sem0)
    cp1 = pltpu.async_copy(emb1_hbm.at[idx1_v], rows1_v, sem1)
    cp0.wait()
    pltpu.sync_copy(rows0_v, out_hbm.at[0, pl.ds(base, _BPW)])
    cp1.wait()
    pltpu.sync_copy(rows1_v, out_hbm.at[1, pl.ds(base, _BPW)])


def kernel(idx0, idx1, emb0, emb1):
    return _embed_sc(idx0, idx1, emb0, emb1)


# speedup vs baseline: 1.4452x; 1.4452x over previous
import functools

import jax
import jax.numpy as jnp
from jax import lax
from jax.experimental import pallas as pl
from jax.experimental.pallas import tpu as pltpu
from jax.experimental.pallas import tpu_sc as plsc

VOCAB = 100000
EMB_DIM = 64
BATCH = 4096
LANGNUM = 2

_info = plsc.get_sparse_core_info()
_NC, _NS, _NL = _info.num_cores, _info.num_subcores, _info.num_lanes
_NW = _NC * _NS
_BPW = BATCH // _NW

_mesh = plsc.VectorSubcoreMesh(core_axis_name="c", subcore_axis_name="s")


@functools.partial(
    pl.kernel,
    mesh=_mesh,
    out_type=jax.ShapeDtypeStruct((LANGNUM, BATCH, EMB_DIM), jnp.float32),
    scratch_types=[
        pltpu.VMEM((_BPW,), jnp.int32),
        pltpu.VMEM((_BPW,), jnp.int32),
        pltpu.VMEM((_BPW, EMB_DIM), jnp.float32),
        pltpu.VMEM((_BPW, EMB_DIM), jnp.float32),
        pltpu.SemaphoreType.DMA,
        pltpu.SemaphoreType.DMA,
    ],
)
def _embed_sc(idx0_hbm, idx1_hbm, emb0_hbm, emb1_hbm, out_hbm,
              idx0_v, idx1_v, rows0_v, rows1_v, sem0, sem1):
    wid = lax.axis_index("s") * _NC + lax.axis_index("c")
    base = wid * _BPW
    pltpu.sync_copy(idx0_hbm.at[pl.ds(base, _BPW)], idx0_v)
    pltpu.sync_copy(idx1_hbm.at[pl.ds(base, _BPW)], idx1_v)

    def make_fire(idx_v, table, rows_v, sem):
        def fire(j, _):
            vec = idx_v[pl.ds(j * _NL, _NL)]
            for t in range(_NL):
                pltpu.make_async_copy(
                    table.at[pl.ds(vec[t], 1), :],
                    rows_v.at[pl.ds(j * _NL + t, 1), :], sem).start()
            return 0
        return fire

    lax.fori_loop(0, _BPW // _NL, make_fire(idx0_v, emb0_hbm, rows0_v, sem0), 0)
    lax.fori_loop(0, _BPW // _NL, make_fire(idx1_v, emb1_hbm, rows1_v, sem1), 0)
    pltpu.make_async_copy(
        emb0_hbm.at[pl.ds(0, _BPW), :], rows0_v, sem0).wait()
    pltpu.sync_copy(rows0_v, out_hbm.at[0, pl.ds(base, _BPW)])
    pltpu.make_async_copy(
        emb1_hbm.at[pl.ds(0, _BPW), :], rows1_v, sem1).wait()
    pltpu.sync_copy(rows1_v, out_hbm.at[1, pl.ds(base, _BPW)])


def kernel(idx0, idx1, emb0, emb1):
    return _embed_sc(idx0, idx1, emb0, emb1)


# speedup vs baseline: 1.4471x; 1.0013x over previous
import functools

import jax
import jax.numpy as jnp
from jax import lax
from jax.experimental import pallas as pl
from jax.experimental.pallas import tpu as pltpu
from jax.experimental.pallas import tpu_sc as plsc

VOCAB = 100000
EMB_DIM = 64
BATCH = 4096
LANGNUM = 2

_info = plsc.get_sparse_core_info()
_NC, _NS, _NL = _info.num_cores, _info.num_subcores, _info.num_lanes
_NW = _NC * _NS
_BPW = BATCH // _NW

_mesh = plsc.VectorSubcoreMesh(core_axis_name="c", subcore_axis_name="s")


@functools.partial(
    pl.kernel,
    mesh=_mesh,
    out_type=jax.ShapeDtypeStruct((LANGNUM, BATCH, EMB_DIM), jnp.float32),
    scratch_types=[
        pltpu.VMEM((_BPW,), jnp.int32),
        pltpu.VMEM((_BPW,), jnp.int32),
        pltpu.VMEM((_BPW, EMB_DIM), jnp.float32),
        pltpu.VMEM((_BPW, EMB_DIM), jnp.float32),
        pltpu.SemaphoreType.DMA,
        pltpu.SemaphoreType.DMA,
    ],
    compiler_params=pltpu.CompilerParams(
        disable_bounds_checks=True,
        disable_semaphore_checks=True,
    ),
)
def _embed_sc(idx0_hbm, idx1_hbm, emb0_hbm, emb1_hbm, out_hbm,
              idx0_v, idx1_v, rows0_v, rows1_v, sem0, sem1):
    wid = lax.axis_index("s") * _NC + lax.axis_index("c")
    base = wid * _BPW
    pltpu.sync_copy(idx0_hbm.at[pl.ds(base, _BPW)], idx0_v)
    pltpu.sync_copy(idx1_hbm.at[pl.ds(base, _BPW)], idx1_v)

    def make_fire(idx_v, table, rows_v, sem):
        def fire(j, _):
            vec = idx_v[pl.ds(j * _NL, _NL)]
            for t in range(_NL):
                pltpu.make_async_copy(
                    table.at[pl.ds(vec[t], 1), :],
                    rows_v.at[pl.ds(j * _NL + t, 1), :], sem).start()
            return 0
        return fire

    lax.fori_loop(0, _BPW // _NL, make_fire(idx0_v, emb0_hbm, rows0_v, sem0), 0)
    lax.fori_loop(0, _BPW // _NL, make_fire(idx1_v, emb1_hbm, rows1_v, sem1), 0)
    pltpu.make_async_copy(
        emb0_hbm.at[pl.ds(0, _BPW), :], rows0_v, sem0).wait()
    pltpu.sync_copy(rows0_v, out_hbm.at[0, pl.ds(base, _BPW)])
    pltpu.make_async_copy(
        emb1_hbm.at[pl.ds(0, _BPW), :], rows1_v, sem1).wait()
    pltpu.sync_copy(rows1_v, out_hbm.at[1, pl.ds(base, _BPW)])


def kernel(idx0, idx1, emb0, emb1):
    return _embed_sc(idx0, idx1, emb0, emb1)
